# baseline (speedup 1.0000x reference)
import functools

import jax
import jax.numpy as jnp
from jax import lax
from jax.experimental import pallas as pl
from jax.experimental.pallas import tpu as pltpu
from jax.experimental.pallas import tpu_sc as plsc

D = 64
V = 1000000
VP = 1000064
B = 16384 * 50
NW = 32
CHUNK = 128
NTASK = B // CHUNK
TPW = NTASK // NW
BPW = TPW * CHUNK
NBUF = 4
NSTG = 2
SP = CHUNK + 1

NBLK = V // CHUNK
BPW1 = 246

_mesh = plsc.VectorSubcoreMesh(core_axis_name="c", subcore_axis_name="s")


@functools.partial(
    pl.kernel,
    mesh=_mesh,
    out_type=jax.ShapeDtypeStruct((V, CHUNK), jnp.float32),
    scratch_types=[
        pltpu.VMEM((2, D, CHUNK), jnp.float32),
        pltpu.VMEM((2, CHUNK, SP), jnp.float32),
        pltpu.SemaphoreType.DMA,
        pltpu.SemaphoreType.DMA,
    ],
    compiler_params=pltpu.CompilerParams(
        use_tc_tiling_on_sc=True, needs_layout_passes=False
    ),
)
def _transpose_sc(tt_hbm, tail_hbm, t2_hbm, buf_v, stg_v, rsem, wsem):
    wid = lax.axis_index("s") * 2 + lax.axis_index("c")

    def rb(k):
        return jnp.minimum(wid * BPW1 + k, NBLK - 1)

    def r_copy(k, b):
        return pltpu.make_async_copy(
            tt_hbm.at[:, pl.ds(rb(k) * CHUNK, CHUNK)], buf_v.at[b], rsem
        )

    def w_copy(k, s):
        return pltpu.make_async_copy(
            stg_v.at[s, :, pl.ds(0, CHUNK)],
            t2_hbm.at[pl.ds(rb(k) * CHUNK, CHUNK), :],
            wsem,
        )

    def transpose(b, s, nq):
        @plsc.parallel_loop(0, D, unroll=4)
        def d_body(d):
            iota = lax.iota(jnp.int32, 16)
            dvec = iota * 0 + d
            for q in range(nq):
                v = buf_v[b, d, pl.ds(q * 16, 16)]
                plsc.store_scatter(stg_v.at[s], [iota + q * 16, dvec], v)

    def step(k, b, wait_prev, start_next):
        r_copy(k, b).wait()
        if wait_prev:
            w_copy(k - 2, b).wait()
        transpose(b, b, CHUNK // 16)
        w_copy(k, b).start()
        if start_next:
            r_copy(k + 2, b).start()

    for b in range(2):
        r_copy(b, b).start()
    for b in range(2):
        step(b, b, wait_prev=False, start_next=True)

    def round_body(r, _):
        for b in range(2):
            step(r * 2 + b, b, wait_prev=True, start_next=True)
        return 0

    lax.fori_loop(1, BPW1 // 2 - 1, round_body, 0)

    for b in range(2):
        step(BPW1 - 2 + b, b, wait_prev=True, start_next=False)
    for b in range(2):
        w_copy(BPW1 - 2 + b, b).wait()

    @pl.when(wid == NW - 1)
    def _tail():
        pltpu.sync_copy(tail_hbm, t2_hbm.at[pl.ds(NBLK * CHUNK, V % CHUNK), :])


@functools.partial(
    pl.kernel,
    mesh=_mesh,
    out_type=jax.ShapeDtypeStruct((NTASK * 8, 8, CHUNK), jnp.float32),
    scratch_types=[
        pltpu.VMEM((BPW,), jnp.int32),
        pltpu.VMEM((NBUF, CHUNK, 2 * D), jnp.float32),
        pltpu.VMEM((NSTG, D, SP), jnp.float32),
        pltpu.SemaphoreType.DMA,
        pltpu.SemaphoreType.DMA,
    ],
    compiler_params=pltpu.CompilerParams(
        use_tc_tiling_on_sc=False, needs_layout_passes=False
    ),
)
def _gather_sc(table_hbm, idx_hbm, out_hbm, idx_v, rows_v, stg_v, gsem, ssem):
    wid = lax.axis_index("s") * 2 + lax.axis_index("c")
    t0 = wid * TPW

    pltpu.sync_copy(idx_hbm.at[pl.ds(wid * BPW, BPW)], idx_v)

    def g_copy(lt, b):
        return pltpu.make_async_copy(
            table_hbm.at[idx_v.at[pl.ds(lt * CHUNK, CHUNK)]], rows_v.at[b], gsem
        )

    def s_copy(lt, td, p):
        t = t0 + lt
        row = (t // CHUNK) * 1024 + td * CHUNK + (t % CHUNK)
        return pltpu.make_async_copy(
            stg_v.at[p, pl.ds(td * 8, 8), pl.ds(0, CHUNK)], out_hbm.at[row], ssem
        )

    def transpose(b, p):
        @plsc.parallel_loop(0, CHUNK, unroll=4)
        def bb_body(bb):
            iota = lax.iota(jnp.int32, 16)
            bbvec = iota * 0 + bb
            for q in range(D // 16):
                v = rows_v[b, bb, pl.ds(q * 16, 16)]
                plsc.store_scatter(stg_v.at[p], [iota + q * 16, bbvec], v)

    def step(lt, b, p, wait_prev, start_next):
        g_copy(lt, b).wait()
        if wait_prev:
            for td in range(8):
                s_copy(lt - NSTG, td, p).wait()
        transpose(b, p)
        for td in range(8):
            s_copy(lt, td, p).start()
        if start_next:
            g_copy(lt + NBUF, b).start()

    for b in range(NBUF):
        g_copy(b, b).start()

    for b in range(NBUF):
        step(b, b, b % NSTG, wait_prev=(b >= NSTG), start_next=True)

    def round_body(r, _):
        for b in range(NBUF):
            lt = r * NBUF + b
            step(lt, b, b % NSTG, wait_prev=True, start_next=True)
        return 0

    lax.fori_loop(1, TPW // NBUF - 1, round_body, 0)

    last = TPW - NBUF
    for b in range(NBUF):
        step(last + b, b, b % NSTG, wait_prev=True, start_next=False)
    for b in range(NSTG):
        for td in range(8):
            s_copy(TPW - NSTG + b, td, b % NSTG).wait()


def kernel(input0, input1):
    idx = input1.astype(jnp.int32).swapaxes(0, 1).reshape(B)
    tail = jnp.pad(input0[V - V % CHUNK:, :], ((0, 0), (0, D)))
    t2 = _transpose_sc(input0.swapaxes(0, 1), tail)
    out = _gather_sc(t2, idx)
    return (
        out.reshape(50, 8, 128, 8, 128)
        .transpose(2, 4, 0, 1, 3)
        .reshape(16384, 50, 64)
    )


# speedup vs baseline: 1.4604x; 1.4604x over previous
import functools

import jax
import jax.numpy as jnp
from jax import lax
from jax.experimental import pallas as pl
from jax.experimental.pallas import tpu as pltpu
from jax.experimental.pallas import tpu_sc as plsc

D = 64
B = 16384 * 50
NW = 32
CHUNK = 128
NTASK = B // CHUNK
TPW = NTASK // NW
BPW = TPW * CHUNK
NBUF = 4
NSTG = 2
SP = CHUNK + 1

_mesh = plsc.VectorSubcoreMesh(core_axis_name="c", subcore_axis_name="s")


@functools.partial(
    pl.kernel,
    mesh=_mesh,
    out_type=jax.ShapeDtypeStruct((NTASK * 8, 8, CHUNK), jnp.float32),
    scratch_types=[
        pltpu.VMEM((BPW,), jnp.int32),
        pltpu.VMEM((NBUF, CHUNK, D), jnp.float32),
        pltpu.VMEM((NSTG, D, SP), jnp.float32),
        pltpu.SemaphoreType.DMA,
        pltpu.SemaphoreType.DMA,
    ],
    compiler_params=pltpu.CompilerParams(
        use_tc_tiling_on_sc=False, needs_layout_passes=False
    ),
)
def _gather_sc(table_hbm, idx_hbm, out_hbm, idx_v, rows_v, stg_v, gsem, ssem):
    wid = lax.axis_index("s") * 2 + lax.axis_index("c")
    t0 = wid * TPW

    pltpu.sync_copy(idx_hbm.at[pl.ds(wid * BPW, BPW)], idx_v)

    def g_copy(lt, b):
        return pltpu.make_async_copy(
            table_hbm.at[idx_v.at[pl.ds(lt * CHUNK, CHUNK)]], rows_v.at[b], gsem
        )

    def s_copy(lt, td, p):
        t = t0 + lt
        row = (t // CHUNK) * 1024 + td * CHUNK + (t % CHUNK)
        return pltpu.make_async_copy(
            stg_v.at[p, pl.ds(td * 8, 8), pl.ds(0, CHUNK)], out_hbm.at[row], ssem
        )

    def transpose(b, p):
        @plsc.parallel_loop(0, CHUNK, unroll=4)
        def bb_body(bb):
            iota = lax.iota(jnp.int32, 16)
            bbvec = iota * 0 + bb
            for q in range(D // 16):
                v = rows_v[b, bb, pl.ds(q * 16, 16)]
                plsc.store_scatter(stg_v.at[p], [iota + q * 16, bbvec], v)

    def step(lt, b, p, wait_prev, start_next):
        g_copy(lt, b).wait()
        if wait_prev:
            for td in range(8):
                s_copy(lt - NSTG, td, p).wait()
        transpose(b, p)
        for td in range(8):
            s_copy(lt, td, p).start()
        if start_next:
            g_copy(lt + NBUF, b).start()

    for b in range(NBUF):
        g_copy(b, b).start()

    for b in range(NBUF):
        step(b, b, b % NSTG, wait_prev=(b >= NSTG), start_next=True)

    def round_body(r, _):
        for b in range(NBUF):
            lt = r * NBUF + b
            step(lt, b, b % NSTG, wait_prev=True, start_next=True)
        return 0

    lax.fori_loop(1, TPW // NBUF - 1, round_body, 0)

    last = TPW - NBUF
    for b in range(NBUF):
        step(last + b, b, b % NSTG, wait_prev=True, start_next=False)
    for b in range(NSTG):
        for td in range(8):
            s_copy(TPW - NSTG + b, td, b % NSTG).wait()


def kernel(input0, input1):
    idx = input1.astype(jnp.int32).swapaxes(0, 1).reshape(B)
    out = _gather_sc(input0, idx)
    return (
        out.reshape(50, 8, 128, 8, 128)
        .transpose(2, 4, 0, 1, 3)
        .reshape(16384, 50, 64)
    )
